# baseline (speedup 1.0000x reference)
import functools

import jax
import jax.numpy as jnp
from jax import lax
from jax.experimental import pallas as pl
from jax.experimental.pallas import tpu as pltpu
from jax.experimental.pallas import tpu_sc as plsc

K_BUCKETS = 1000
EMBED_DIM = 128
BATCH = 16384

_NC = 2
_NS = 16
_NW = _NC * _NS
_B_PER_W = BATCH // _NW

_STAGE = 64

_mesh = plsc.VectorSubcoreMesh(core_axis_name="c", subcore_axis_name="s")


@functools.partial(
    pl.kernel,
    mesh=_mesh,
    out_type=jax.ShapeDtypeStruct((BATCH, EMBED_DIM), jnp.float32),
    scratch_types=[
        pltpu.VMEM((_B_PER_W,), jnp.int32),
        pltpu.VMEM((_B_PER_W, EMBED_DIM), jnp.float32),
        pltpu.VMEM_SHARED((K_BUCKETS, EMBED_DIM), jnp.float32),
        pltpu.SemaphoreType.DMA,
    ],
)
def _gather_kernel(ids_hbm, table_hbm, out_hbm, idx_v, rows_v, tbl_sp, sem):
    cid = lax.axis_index("c")
    sid = lax.axis_index("s")
    wid = sid * _NC + cid
    base = wid * _B_PER_W

    @pl.when(sid < _NS - 1)
    def _():
        row0 = sid * _STAGE
        pltpu.sync_copy(table_hbm.at[pl.ds(row0, _STAGE)],
                        tbl_sp.at[pl.ds(row0, _STAGE)])

    @pl.when(sid == _NS - 1)
    def _():
        last = (_NS - 1) * _STAGE
        pltpu.sync_copy(table_hbm.at[pl.ds(last, K_BUCKETS - last)],
                        tbl_sp.at[pl.ds(last, K_BUCKETS - last)])

    pltpu.sync_copy(ids_hbm.at[pl.ds(base, _B_PER_W)], idx_v)
    plsc.subcore_barrier()

    pltpu.async_copy(tbl_sp.at[idx_v], rows_v, sem).wait()
    pltpu.sync_copy(rows_v, out_hbm.at[pl.ds(base, _B_PER_W)])


def kernel(ids, embed_weight):
    return _gather_kernel(ids.astype(jnp.int32), embed_weight)
